# baseline (speedup 1.0000x reference)
import functools

import jax
import jax.numpy as jnp
from jax import lax
from jax.experimental import pallas as pl
from jax.experimental.pallas import tpu as pltpu
from jax.experimental.pallas import tpu_sc as plsc

DIM = 256
BN = 1024
BK = 512
BT = 512
BC = 1024
GC = 128


def _dist_argmin_body(xt_ref, e_ref, xsq_ref, esq_ref, idx_ref,
                      best_ref, bidx_ref):
    k = pl.program_id(1)
    nk = pl.num_programs(1)

    @pl.when(k == 0)
    def _init():
        best_ref[...] = jnp.full(best_ref.shape, -jnp.inf, best_ref.dtype)
        bidx_ref[...] = jnp.zeros(bidx_ref.shape, bidx_ref.dtype)

    eb = e_ref[...]
    xtb = xt_ref[...]
    mt = lax.dot_general(eb, xtb, (((0,), (0,)), ((), ())),
                         preferred_element_type=jnp.float32,
                         precision=lax.Precision.HIGHEST)
    d = (xsq_ref[...] - 2.0 * mt) + esq_ref[...]
    nd = -d

    tmax = jnp.max(nd, axis=0, keepdims=True)
    sub = lax.broadcasted_iota(jnp.int32, nd.shape, 0)
    targ = jnp.min(jnp.where(nd == tmax, sub, BK), axis=0, keepdims=True)
    better = tmax > best_ref[...]
    bidx_ref[...] = jnp.where(better, k * BK + targ, bidx_ref[...])
    best_ref[...] = jnp.where(better, tmax, best_ref[...])

    @pl.when(k == nk - 1)
    def _fin():
        idx_ref[...] = bidx_ref[...][None]


def _argmin_indices(xt, emb, xsq, esq):
    n, k = xt.shape[1], emb.shape[1]
    out = pl.pallas_call(
        _dist_argmin_body,
        grid=(n // BN, k // BK),
        in_specs=[
            pl.BlockSpec((DIM, BN), lambda i, j: (0, i)),
            pl.BlockSpec((DIM, BK), lambda i, j: (0, j)),
            pl.BlockSpec((1, BN), lambda i, j: (0, i)),
            pl.BlockSpec((BK, 1), lambda i, j: (j, 0)),
        ],
        out_specs=pl.BlockSpec((1, 1, BN), lambda i, j: (i, 0, 0)),
        out_shape=jax.ShapeDtypeStruct((n // BN, 1, BN), jnp.int32),
        scratch_shapes=[
            pltpu.VMEM((1, BN), jnp.float32),
            pltpu.VMEM((1, BN), jnp.int32),
        ],
        compiler_params=pltpu.CompilerParams(
            dimension_semantics=("parallel", "arbitrary"),
        ),
    )(xt, emb, xsq, esq)
    return out.reshape(n)


def _transpose_body(e_ref, et_ref):
    et_ref[...] = e_ref[...].T


def _transpose(emb):
    k = emb.shape[1]
    return pl.pallas_call(
        _transpose_body,
        grid=(k // BT,),
        in_specs=[pl.BlockSpec((DIM, BT), lambda j: (0, j))],
        out_specs=pl.BlockSpec((BT, DIM), lambda j: (j, 0)),
        out_shape=jax.ShapeDtypeStruct((k, DIM), jnp.float32),
    )(emb)


def _gather_rows(emb_t, idx):
    info = plsc.get_sparse_core_info()
    nw = info.num_cores * info.num_subcores
    b = idx.shape[0]
    b_per_w = b // nw
    n_chunks = b_per_w // GC
    mesh = plsc.VectorSubcoreMesh(core_axis_name="c", subcore_axis_name="s")

    @functools.partial(
        pl.kernel,
        out_type=jax.ShapeDtypeStruct((b, DIM), jnp.float32),
        mesh=mesh,
        scratch_types=[
            pltpu.VMEM((GC,), jnp.int32),
            pltpu.VMEM((GC, DIM), jnp.float32),
            pltpu.SemaphoreType.DMA,
        ],
    )
    def gk(table_hbm, idx_hbm, out_hbm, idx_v, rows_v, sem):
        wid = lax.axis_index("s") * info.num_cores + lax.axis_index("c")
        base = wid * b_per_w

        def chunk(c, carry):
            off = base + c * GC
            pltpu.sync_copy(idx_hbm.at[pl.ds(off, GC)], idx_v)
            pltpu.async_copy(table_hbm.at[idx_v], rows_v, sem).wait()
            pltpu.sync_copy(rows_v, out_hbm.at[pl.ds(off, GC)])
            return carry

        lax.fori_loop(0, n_chunks, chunk, 0)

    return gk(emb_t, idx)


def _finalize_body(x_ref, q_ref, qst_ref, loss_ref, acc_ref):
    i = pl.program_id(0)
    ni = pl.num_programs(0)

    @pl.when(i == 0)
    def _init():
        acc_ref[0] = 0.0

    xb = x_ref[...]
    qb = q_ref[...]
    qst_ref[...] = xb + (qb - xb)
    diff = xb - qb
    acc_ref[0] += jnp.sum(diff * diff)

    @pl.when(i == ni - 1)
    def _fin():
        denom = float(x_ref.shape[0] * ni * x_ref.shape[1])
        loss_ref[...] = (acc_ref[0] / denom)[None, None]


def _finalize(x2d, q):
    n = x2d.shape[0]
    return pl.pallas_call(
        _finalize_body,
        grid=(n // BC,),
        in_specs=[
            pl.BlockSpec((BC, DIM), lambda i: (i, 0)),
            pl.BlockSpec((BC, DIM), lambda i: (i, 0)),
        ],
        out_specs=[
            pl.BlockSpec((BC, DIM), lambda i: (i, 0)),
            pl.BlockSpec((1, 1), lambda i: (0, 0)),
        ],
        out_shape=[
            jax.ShapeDtypeStruct((n, DIM), jnp.float32),
            jax.ShapeDtypeStruct((1, 1), jnp.float32),
        ],
        scratch_shapes=[pltpu.SMEM((1,), jnp.float32)],
    )(x2d, q)


def kernel(x, embedding):
    bsz, seq, dim = x.shape
    x2d = x.reshape(-1, dim)
    xt = x2d.T
    xsq = jnp.sum(x2d ** 2, axis=1)[None, :]
    esq = jnp.sum(embedding ** 2, axis=0)[:, None]
    idx = _argmin_indices(xt, embedding, xsq, esq)
    emb_t = _transpose(embedding)
    q = _gather_rows(emb_t, idx)
    qst2d, loss2d = _finalize(x2d, q)
    quantize_st = qst2d.reshape(x.shape)
    loss = loss2d.reshape(())
    indices = idx.reshape(bsz, seq)
    return (quantize_st, loss, indices, loss)
